# baseline (speedup 1.0000x reference)
import jax
import jax.numpy as jnp
from jax.experimental import pallas as pl
from jax.experimental.pallas import tpu as pltpu

_BLK = 1024


def _dma_body(tab_hbm, out_hbm, buf, rsem, wsem):
    batch, seq_len, embed_dim = out_hbm.shape
    n = seq_len // _BLK

    def read(i):
        return pltpu.make_async_copy(
            tab_hbm.at[pl.ds(i * _BLK, _BLK), :], buf.at[i % 2], rsem.at[i % 2])

    def write(i, b):
        return pltpu.make_async_copy(
            buf.at[i % 2], out_hbm.at[b, pl.ds(i * _BLK, _BLK), :],
            wsem.at[i % 2])

    read(0).start()
    for i in range(n):
        read(i).wait()
        for b in range(batch):
            write(i, b).start()
        if i + 1 < n:
            if i >= 1:
                for b in range(batch):
                    write(i - 1, b).wait()
            read(i + 1).start()
    if n >= 2:
        for b in range(batch):
            write(n - 2, b).wait()
    for b in range(batch):
        write(n - 1, b).wait()


def kernel(x, position_embedding):
    batch, seq_len, embed_dim = x.shape
    return pl.pallas_call(
        _dma_body,
        in_specs=[pl.BlockSpec(memory_space=pltpu.MemorySpace.HBM)],
        out_specs=pl.BlockSpec(memory_space=pltpu.MemorySpace.HBM),
        out_shape=jax.ShapeDtypeStruct((batch, seq_len, embed_dim),
                                       position_embedding.dtype),
        scratch_shapes=[
            pltpu.VMEM((2, _BLK, embed_dim), position_embedding.dtype),
            pltpu.SemaphoreType.DMA((2,)),
            pltpu.SemaphoreType.DMA((2,)),
        ],
    )(position_embedding[:seq_len])


# speedup vs baseline: 1.0165x; 1.0165x over previous
import jax
import jax.numpy as jnp
from jax.experimental import pallas as pl
from jax.experimental.pallas import tpu as pltpu

_BLK = 4096


def _dma_body(tab_hbm, out_hbm, buf, rsem, wsem):
    batch, seq_len, embed_dim = out_hbm.shape
    n = seq_len // _BLK

    def read(i):
        return pltpu.make_async_copy(
            tab_hbm.at[pl.ds(i * _BLK, _BLK), :], buf.at[i % 2], rsem.at[i % 2])

    def write(i, b):
        return pltpu.make_async_copy(
            buf.at[i % 2], out_hbm.at[b, pl.ds(i * _BLK, _BLK), :],
            wsem.at[i % 2])

    read(0).start()
    for i in range(n):
        read(i).wait()
        for b in range(batch):
            write(i, b).start()
        if i + 1 < n:
            if i >= 1:
                for b in range(batch):
                    write(i - 1, b).wait()
            read(i + 1).start()
    if n >= 2:
        for b in range(batch):
            write(n - 2, b).wait()
    for b in range(batch):
        write(n - 1, b).wait()


def kernel(x, position_embedding):
    batch, seq_len, embed_dim = x.shape
    return pl.pallas_call(
        _dma_body,
        in_specs=[pl.BlockSpec(memory_space=pltpu.MemorySpace.HBM)],
        out_specs=pl.BlockSpec(memory_space=pltpu.MemorySpace.HBM),
        out_shape=jax.ShapeDtypeStruct((batch, seq_len, embed_dim),
                                       position_embedding.dtype),
        scratch_shapes=[
            pltpu.VMEM((2, _BLK, embed_dim), position_embedding.dtype),
            pltpu.SemaphoreType.DMA((2,)),
            pltpu.SemaphoreType.DMA((2,)),
        ],
    )(position_embedding[:seq_len])
